# baseline (speedup 1.0000x reference)
import functools

import jax
import jax.numpy as jnp
from jax import lax
from jax.experimental import pallas as pl
from jax.experimental.pallas import tpu as pltpu
from jax.experimental.pallas import tpu_sc as plsc

N_NODES = 10000
N_EDGES = 320000
D_FEAT = 128

NW = 32
E_PER_W = N_EDGES // NW
CHUNK = 80
NCHUNK = E_PER_W // CHUNK
GROUPS = CHUNK // 16

_mesh = plsc.VectorSubcoreMesh(core_axis_name="c", subcore_axis_name="s")


@functools.partial(
    pl.kernel,
    out_type=jax.ShapeDtypeStruct((N_EDGES,), jnp.float32),
    mesh=_mesh,
    compiler_params=pltpu.CompilerParams(needs_layout_passes=False),
    scratch_types=[
        pltpu.VMEM((E_PER_W,), jnp.int32),
        pltpu.VMEM((E_PER_W,), jnp.int32),
        pltpu.VMEM((CHUNK, D_FEAT), jnp.float32),
        pltpu.VMEM((CHUNK, D_FEAT), jnp.float32),
        pltpu.VMEM((CHUNK, D_FEAT), jnp.float32),
        pltpu.VMEM((CHUNK, D_FEAT), jnp.float32),
        pltpu.VMEM((E_PER_W,), jnp.float32),
        pltpu.VMEM((16 * 17,), jnp.float32),
        pltpu.SemaphoreType.DMA,
        pltpu.SemaphoreType.DMA,
        pltpu.SemaphoreType.DMA,
        pltpu.SemaphoreType.DMA,
    ],
)
def _decode_probs(z_hbm, row_hbm, col_hbm, out_hbm,
                  ridx, cidx, a0, b0, a1, b1, obuf, tbuf,
                  sem_a0, sem_b0, sem_a1, sem_b1):
    wid = lax.axis_index("s") * 2 + lax.axis_index("c")
    base = wid * E_PER_W
    lanes = lax.iota(jnp.int32, 16)
    scat_idx = [lanes + (e * 17) for e in range(16)]
    col_idx = [(lanes * 17) + j for j in range(16)]

    pltpu.sync_copy(row_hbm.at[pl.ds(base, E_PER_W)], ridx)
    pltpu.sync_copy(col_hbm.at[pl.ds(base, E_PER_W)], cidx)

    def gather(ci, abuf, bbuf, sa, sb):
        sl = pl.ds(ci * CHUNK, CHUNK)
        pltpu.async_copy(z_hbm.at[ridx.at[sl]], abuf, sa)
        pltpu.async_copy(z_hbm.at[cidx.at[sl]], bbuf, sb)

    def wait(abuf, bbuf, sa, sb):
        pltpu.make_async_copy(z_hbm.at[ridx.at[pl.ds(0, CHUNK)]], abuf, sa).wait()
        pltpu.make_async_copy(z_hbm.at[cidx.at[pl.ds(0, CHUNK)]], bbuf, sb).wait()

    def compute(ci, abuf, bbuf):
        def group_body(g, carry):
            base_e = g * 16
            out_off = ci * CHUNK + base_e
            @plsc.parallel_loop(0, 16, 1, unroll=8)
            def _edge_body(e):
                row = base_e + e
                acc = (abuf[row, pl.ds(0, 16)] * bbuf[row, pl.ds(0, 16)])
                for k in range(1, D_FEAT // 16):
                    acc = acc + (abuf[row, pl.ds(k * 16, 16)]
                                 * bbuf[row, pl.ds(k * 16, 16)])
                plsc.store_scatter(tbuf, [lanes + e * 17], acc)
            dot = plsc.load_gather(tbuf, [col_idx[0]])
            for j in range(1, 16):
                dot = dot + plsc.load_gather(tbuf, [col_idx[j]])
            obuf[pl.ds(out_off, 16)] = 1.0 / (1.0 + jnp.exp(-dot))
            return carry
        lax.fori_loop(0, GROUPS, group_body, 0)

    gather(0, a0, b0, sem_a0, sem_b0)

    def pair_body(i, carry):
        c0 = 2 * i
        gather(c0 + 1, a1, b1, sem_a1, sem_b1)
        wait(a0, b0, sem_a0, sem_b0)
        compute(c0, a0, b0)
        gather(c0 + 2, a0, b0, sem_a0, sem_b0)
        wait(a1, b1, sem_a1, sem_b1)
        compute(c0 + 1, a1, b1)
        return carry

    lax.fori_loop(0, (NCHUNK - 1) // 2, pair_body, 0)
    wait(a0, b0, sem_a0, sem_b0)
    compute(NCHUNK - 1, a0, b0)

    pltpu.sync_copy(obuf, out_hbm.at[pl.ds(base, E_PER_W)])


def kernel(z, edge_index):
    edge_index = edge_index.astype(jnp.int32)
    probs = _decode_probs(z, edge_index[0], edge_index[1])
    labels = jnp.ones((N_EDGES,), dtype=jnp.float32)
    return probs, labels


# speedup vs baseline: 1.0382x; 1.0382x over previous
import functools

import jax
import jax.numpy as jnp
from jax import lax
from jax.experimental import pallas as pl
from jax.experimental.pallas import tpu as pltpu
from jax.experimental.pallas import tpu_sc as plsc

N_NODES = 10000
N_EDGES = 320000
D_FEAT = 128

NW = 32
E_PER_W = N_EDGES // NW
CHUNK = 80
NCHUNK = E_PER_W // CHUNK
GROUPS = CHUNK // 16

_mesh = plsc.VectorSubcoreMesh(core_axis_name="c", subcore_axis_name="s")


@functools.partial(
    pl.kernel,
    out_type=jax.ShapeDtypeStruct((N_EDGES,), jnp.float32),
    mesh=_mesh,
    compiler_params=pltpu.CompilerParams(needs_layout_passes=False),
    scratch_types=[
        pltpu.VMEM((E_PER_W,), jnp.int32),
        pltpu.VMEM((E_PER_W,), jnp.int32),
        pltpu.VMEM((CHUNK, D_FEAT), jnp.float32),
        pltpu.VMEM((CHUNK, D_FEAT), jnp.float32),
        pltpu.VMEM((CHUNK, D_FEAT), jnp.float32),
        pltpu.VMEM((CHUNK, D_FEAT), jnp.float32),
        pltpu.VMEM((E_PER_W,), jnp.float32),
        pltpu.VMEM((CHUNK * 17,), jnp.float32),
        pltpu.SemaphoreType.DMA,
        pltpu.SemaphoreType.DMA,
        pltpu.SemaphoreType.DMA,
        pltpu.SemaphoreType.DMA,
    ],
)
def _decode_probs(z_hbm, row_hbm, col_hbm, out_hbm,
                  ridx, cidx, a0, b0, a1, b1, obuf, tbuf,
                  sem_a0, sem_b0, sem_a1, sem_b1):
    wid = lax.axis_index("s") * 2 + lax.axis_index("c")
    base = wid * E_PER_W
    lanes = lax.iota(jnp.int32, 16)
    scat_idx = [lanes + (e * 17) for e in range(16)]
    col_idx = [(lanes * 17) + j for j in range(16)]

    pltpu.sync_copy(row_hbm.at[pl.ds(base, E_PER_W)], ridx)
    pltpu.sync_copy(col_hbm.at[pl.ds(base, E_PER_W)], cidx)

    def gather(ci, abuf, bbuf, sa, sb):
        sl = pl.ds(ci * CHUNK, CHUNK)
        pltpu.async_copy(z_hbm.at[ridx.at[sl]], abuf, sa)
        pltpu.async_copy(z_hbm.at[cidx.at[sl]], bbuf, sb)

    def wait(abuf, bbuf, sa, sb):
        pltpu.make_async_copy(z_hbm.at[ridx.at[pl.ds(0, CHUNK)]], abuf, sa).wait()
        pltpu.make_async_copy(z_hbm.at[cidx.at[pl.ds(0, CHUNK)]], bbuf, sb).wait()

    def compute(ci, abuf, bbuf):
        @plsc.parallel_loop(0, CHUNK, 1, unroll=4)
        def _edge_body(e):
            acc = (abuf[e, pl.ds(0, 16)] * bbuf[e, pl.ds(0, 16)])
            for k in range(1, D_FEAT // 16):
                acc = acc + (abuf[e, pl.ds(k * 16, 16)]
                             * bbuf[e, pl.ds(k * 16, 16)])
            plsc.store_scatter(tbuf, [lanes + e * 17], acc)

        for g in range(GROUPS):
            tb = g * 16 * 17
            dot = plsc.load_gather(tbuf, [col_idx[0] + tb])
            for j in range(1, 16):
                dot = dot + plsc.load_gather(tbuf, [col_idx[j] + tb])
            obuf[pl.ds(ci * CHUNK + g * 16, 16)] = 1.0 / (1.0 + jnp.exp(-dot))

    gather(0, a0, b0, sem_a0, sem_b0)

    def pair_body(i, carry):
        c0 = 2 * i
        gather(c0 + 1, a1, b1, sem_a1, sem_b1)
        wait(a0, b0, sem_a0, sem_b0)
        compute(c0, a0, b0)
        gather(c0 + 2, a0, b0, sem_a0, sem_b0)
        wait(a1, b1, sem_a1, sem_b1)
        compute(c0 + 1, a1, b1)
        return carry

    lax.fori_loop(0, (NCHUNK - 1) // 2, pair_body, 0)
    wait(a0, b0, sem_a0, sem_b0)
    compute(NCHUNK - 1, a0, b0)

    pltpu.sync_copy(obuf, out_hbm.at[pl.ds(base, E_PER_W)])


def kernel(z, edge_index):
    edge_index = edge_index.astype(jnp.int32)
    probs = _decode_probs(z, edge_index[0], edge_index[1])
    labels = jnp.ones((N_EDGES,), dtype=jnp.float32)
    return probs, labels


# speedup vs baseline: 1.0610x; 1.0220x over previous
import functools

import jax
import jax.numpy as jnp
from jax import lax
from jax.experimental import pallas as pl
from jax.experimental.pallas import tpu as pltpu
from jax.experimental.pallas import tpu_sc as plsc

N_NODES = 10000
N_EDGES = 320000
D_FEAT = 128

NW = 32
E_PER_W = N_EDGES // NW
CHUNK = 80
NCHUNK = E_PER_W // CHUNK
GROUPS = CHUNK // 16

_mesh = plsc.VectorSubcoreMesh(core_axis_name="c", subcore_axis_name="s")


@functools.partial(
    pl.kernel,
    out_type=jax.ShapeDtypeStruct((N_EDGES,), jnp.float32),
    mesh=_mesh,
    compiler_params=pltpu.CompilerParams(needs_layout_passes=False),
    scratch_types=[
        pltpu.VMEM((E_PER_W,), jnp.int32),
        pltpu.VMEM((E_PER_W,), jnp.int32),
        pltpu.VMEM((CHUNK, D_FEAT), jnp.float32),
        pltpu.VMEM((CHUNK, D_FEAT), jnp.float32),
        pltpu.VMEM((CHUNK, D_FEAT), jnp.float32),
        pltpu.VMEM((CHUNK, D_FEAT), jnp.float32),
        pltpu.VMEM((E_PER_W,), jnp.float32),
        pltpu.VMEM((16 * 17,), jnp.float32),
        pltpu.SemaphoreType.DMA,
        pltpu.SemaphoreType.DMA,
        pltpu.SemaphoreType.DMA,
        pltpu.SemaphoreType.DMA,
    ],
)
def _decode_probs(z_hbm, row_hbm, col_hbm, out_hbm,
                  ridx, cidx, a0, b0, a1, b1, obuf, tbuf,
                  sem_a0, sem_b0, sem_a1, sem_b1):
    wid = lax.axis_index("s") * 2 + lax.axis_index("c")
    base = wid * E_PER_W
    lanes = lax.iota(jnp.int32, 16)
    scat_idx = [lanes + (e * 17) for e in range(16)]
    col_idx = [(lanes * 17) + j for j in range(16)]

    pltpu.sync_copy(row_hbm.at[pl.ds(base, E_PER_W)], ridx)
    pltpu.sync_copy(col_hbm.at[pl.ds(base, E_PER_W)], cidx)

    def gather(ci, abuf, bbuf, sa, sb):
        sl = pl.ds(ci * CHUNK, CHUNK)
        pltpu.async_copy(z_hbm.at[ridx.at[sl]], abuf, sa)
        pltpu.async_copy(z_hbm.at[cidx.at[sl]], bbuf, sb)

    def wait(abuf, bbuf, sa, sb):
        pltpu.make_async_copy(z_hbm.at[ridx.at[pl.ds(0, CHUNK)]], abuf, sa).wait()
        pltpu.make_async_copy(z_hbm.at[cidx.at[pl.ds(0, CHUNK)]], bbuf, sb).wait()

    def compute(ci, abuf, bbuf):
        def group_body(g, carry):
            base_e = g * 16
            out_off = ci * CHUNK + base_e
            @plsc.parallel_loop(0, 16, 1, unroll=2)
            def _edge_body(e):
                row = base_e + e
                acc = (abuf[row, pl.ds(0, 16)] * bbuf[row, pl.ds(0, 16)])
                for k in range(1, D_FEAT // 16):
                    acc = acc + (abuf[row, pl.ds(k * 16, 16)]
                                 * bbuf[row, pl.ds(k * 16, 16)])
                plsc.store_scatter(tbuf, [lanes + e * 17], acc)
            dot = plsc.load_gather(tbuf, [col_idx[0]])
            for j in range(1, 16):
                dot = dot + plsc.load_gather(tbuf, [col_idx[j]])
            obuf[pl.ds(out_off, 16)] = 1.0 / (1.0 + jnp.exp(-dot))
            return carry
        lax.fori_loop(0, GROUPS, group_body, 0)

    gather(0, a0, b0, sem_a0, sem_b0)

    def pair_body(i, carry):
        c0 = 2 * i
        gather(c0 + 1, a1, b1, sem_a1, sem_b1)
        wait(a0, b0, sem_a0, sem_b0)
        compute(c0, a0, b0)
        gather(c0 + 2, a0, b0, sem_a0, sem_b0)
        wait(a1, b1, sem_a1, sem_b1)
        compute(c0 + 1, a1, b1)
        return carry

    lax.fori_loop(0, (NCHUNK - 1) // 2, pair_body, 0)
    wait(a0, b0, sem_a0, sem_b0)
    compute(NCHUNK - 1, a0, b0)

    pltpu.sync_copy(obuf, out_hbm.at[pl.ds(base, E_PER_W)])


def kernel(z, edge_index):
    edge_index = edge_index.astype(jnp.int32)
    probs = _decode_probs(z, edge_index[0], edge_index[1])
    labels = jnp.ones((N_EDGES,), dtype=jnp.float32)
    return probs, labels


# speedup vs baseline: 1.0613x; 1.0003x over previous
import functools

import jax
import jax.numpy as jnp
from jax import lax
from jax.experimental import pallas as pl
from jax.experimental.pallas import tpu as pltpu
from jax.experimental.pallas import tpu_sc as plsc

N_NODES = 10000
N_EDGES = 320000
D_FEAT = 128

NW = 32
E_PER_W = N_EDGES // NW
CHUNK = 80
NCHUNK = E_PER_W // CHUNK
GROUPS = CHUNK // 16

_mesh = plsc.VectorSubcoreMesh(core_axis_name="c", subcore_axis_name="s")


@functools.partial(
    pl.kernel,
    out_type=jax.ShapeDtypeStruct((N_EDGES,), jnp.float32),
    mesh=_mesh,
    compiler_params=pltpu.CompilerParams(needs_layout_passes=False),
    scratch_types=[
        pltpu.VMEM((E_PER_W,), jnp.int32),
        pltpu.VMEM((E_PER_W,), jnp.int32),
        pltpu.VMEM((CHUNK, D_FEAT), jnp.float32),
        pltpu.VMEM((CHUNK, D_FEAT), jnp.float32),
        pltpu.VMEM((CHUNK, D_FEAT), jnp.float32),
        pltpu.VMEM((CHUNK, D_FEAT), jnp.float32),
        pltpu.VMEM((E_PER_W,), jnp.float32),
        pltpu.VMEM((16 * 17,), jnp.float32),
        pltpu.SemaphoreType.DMA,
        pltpu.SemaphoreType.DMA,
        pltpu.SemaphoreType.DMA,
        pltpu.SemaphoreType.DMA,
    ],
)
def _decode_probs(z_hbm, row_hbm, col_hbm, out_hbm,
                  ridx, cidx, a0, b0, a1, b1, obuf, tbuf,
                  sem_a0, sem_b0, sem_a1, sem_b1):
    wid = lax.axis_index("s") * 2 + lax.axis_index("c")
    base = wid * E_PER_W
    lanes = lax.iota(jnp.int32, 16)
    col_idx = [(lanes * 17) + j for j in range(16)]

    pltpu.sync_copy(row_hbm.at[pl.ds(base, E_PER_W)], ridx)
    pltpu.sync_copy(col_hbm.at[pl.ds(base, E_PER_W)], cidx)

    def gather(ci, abuf, bbuf, sa, sb):
        sl = pl.ds(ci * CHUNK, CHUNK)
        pltpu.async_copy(z_hbm.at[ridx.at[sl]], abuf, sa)
        pltpu.async_copy(z_hbm.at[cidx.at[sl]], bbuf, sb)

    def wait(abuf, bbuf, sa, sb):
        pltpu.make_async_copy(z_hbm.at[ridx.at[pl.ds(0, CHUNK)]], abuf, sa).wait()
        pltpu.make_async_copy(z_hbm.at[cidx.at[pl.ds(0, CHUNK)]], bbuf, sb).wait()

    def compute(ci, abuf, bbuf):
        def group_body(g, carry):
            base_e = g * 16
            out_off = ci * CHUNK + base_e
            @plsc.parallel_loop(0, 16, 1, unroll=2)
            def _edge_body(e):
                row = base_e + e
                acc = (abuf[row, pl.ds(0, 16)] * bbuf[row, pl.ds(0, 16)])
                for k in range(1, D_FEAT // 16):
                    acc = acc + (abuf[row, pl.ds(k * 16, 16)]
                                 * bbuf[row, pl.ds(k * 16, 16)])
                plsc.store_scatter(tbuf, [lanes + e * 17], acc)
            dot = plsc.load_gather(tbuf, [col_idx[0]])
            for j in range(1, 16):
                dot = dot + plsc.load_gather(tbuf, [col_idx[j]])
            obuf[pl.ds(out_off, 16)] = 1.0 / (1.0 + jnp.exp(-dot))
            return carry
        lax.fori_loop(0, GROUPS, group_body, 0)

    gather(0, a0, b0, sem_a0, sem_b0)

    def pair_body(i, carry):
        c0 = 2 * i
        gather(c0 + 1, a1, b1, sem_a1, sem_b1)
        wait(a0, b0, sem_a0, sem_b0)
        compute(c0, a0, b0)
        gather(c0 + 2, a0, b0, sem_a0, sem_b0)
        wait(a1, b1, sem_a1, sem_b1)
        compute(c0 + 1, a1, b1)
        return carry

    lax.fori_loop(0, (NCHUNK - 1) // 2, pair_body, 0)
    wait(a0, b0, sem_a0, sem_b0)
    compute(NCHUNK - 1, a0, b0)

    pltpu.sync_copy(obuf, out_hbm.at[pl.ds(base, E_PER_W)])


def kernel(z, edge_index):
    edge_index = edge_index.astype(jnp.int32)
    probs = _decode_probs(z, edge_index[0], edge_index[1])
    labels = jnp.ones((N_EDGES,), dtype=jnp.float32)
    return probs, labels
